# baseline (speedup 1.0000x reference)
import functools

import jax
import jax.numpy as jnp
from jax import lax
from jax.experimental import pallas as pl
from jax.experimental.pallas import tpu as pltpu


def _se_body(x_ref, w1_ref, w2_ref, o_ref, *, inv_hw):
    xf = x_ref[...].astype(jnp.float32)
    hw = xf.shape[-1]
    ones = jnp.ones((hw,), jnp.float32)
    pooled = lax.dot_general(
        xf, ones, (((2,), (0,)), ((), ())),
        preferred_element_type=jnp.float32) * inv_hw
    h = lax.dot_general(
        pooled, w1_ref[...], (((1,), (1,)), ((), ())),
        preferred_element_type=jnp.float32)
    h = jnp.maximum(h, 0.0)
    s = jax.nn.sigmoid(lax.dot_general(
        h, w2_ref[...], (((1,), (1,)), ((), ())),
        preferred_element_type=jnp.float32))
    o_ref[...] = (xf * s[:, :, None]).astype(o_ref.dtype)


def _pick_tb(B, per_elem_bytes):
    for cand in range(B, 0, -1):
        if B % cand:
            continue
        steps = B // cand
        if steps % 2 == 0 and cand * per_elem_bytes <= (4 << 20):
            return cand
    return 1


@jax.jit
def kernel(x_nchw, w1, w2):
    B, C, H, W = x_nchw.shape
    HW = H * W
    x = x_nchw.reshape(B, C, HW)

    per_elem_bytes = C * HW * x.dtype.itemsize
    TB = _pick_tb(B, per_elem_bytes)

    out = pl.pallas_call(
        functools.partial(_se_body, inv_hw=1.0 / float(HW)),
        out_shape=jax.ShapeDtypeStruct((B, C, HW), x.dtype),
        grid=(B // TB,),
        in_specs=[
            pl.BlockSpec((TB, C, HW), lambda b: (b, 0, 0)),
            pl.BlockSpec(w1.shape, lambda b: (0, 0)),
            pl.BlockSpec(w2.shape, lambda b: (0, 0)),
        ],
        out_specs=pl.BlockSpec((TB, C, HW), lambda b: (b, 0, 0)),
        compiler_params=pltpu.CompilerParams(
            dimension_semantics=("parallel",),
            vmem_limit_bytes=48 << 20,
        ),
        cost_estimate=pl.CostEstimate(
            flops=int(2 * B * C * HW + 4 * B * C * w1.shape[0]),
            transcendentals=int(B * C),
            bytes_accessed=int(2 * B * C * HW * x.dtype.itemsize),
        ),
    )(x, w1, w2)
    return out.reshape(B, C, H, W)


# speedup vs baseline: 3.0575x; 3.0575x over previous
import functools

import jax
import jax.numpy as jnp
from jax import lax
from jax.experimental import pallas as pl
from jax.experimental.pallas import tpu as pltpu


def _se_body(x_ref, w1_ref, w2t_ref, o_ref, *, inv_hw):
    xf = x_ref[...].astype(jnp.float32)
    pooled = jnp.sum(xf, axis=1) * inv_hw
    h = lax.dot_general(
        pooled, w1_ref[...], (((1,), (1,)), ((), ())),
        preferred_element_type=jnp.float32)
    h = jnp.maximum(h, 0.0)
    s = jax.nn.sigmoid(
        jnp.dot(h, w2t_ref[...], preferred_element_type=jnp.float32))
    o_ref[...] = (xf * s[:, None, :]).astype(o_ref.dtype)


@jax.jit
def kernel(x_nchw, w1, w2):
    B, C, H, W = x_nchw.shape
    HW = H * W
    x = x_nchw.transpose(0, 2, 3, 1).reshape(B, HW, C)
    w2t = w2.T

    TB = 4
    while B % TB:
        TB -= 1

    out = pl.pallas_call(
        functools.partial(_se_body, inv_hw=1.0 / float(HW)),
        out_shape=jax.ShapeDtypeStruct((B, HW, C), x.dtype),
        grid=(B // TB,),
        in_specs=[
            pl.BlockSpec((TB, HW, C), lambda b: (b, 0, 0)),
            pl.BlockSpec(w1.shape, lambda b: (0, 0)),
            pl.BlockSpec(w2t.shape, lambda b: (0, 0)),
        ],
        out_specs=pl.BlockSpec((TB, HW, C), lambda b: (b, 0, 0)),
        compiler_params=pltpu.CompilerParams(
            dimension_semantics=("parallel",),
            vmem_limit_bytes=48 << 20,
        ),
        cost_estimate=pl.CostEstimate(
            flops=int(2 * B * C * HW + 4 * B * C * w1.shape[0]),
            transcendentals=int(B * C),
            bytes_accessed=int(2 * B * C * HW * x.dtype.itemsize),
        ),
    )(x, w1, w2t)
    return out.reshape(B, H, W, C).transpose(0, 3, 1, 2)


# speedup vs baseline: 3.4897x; 1.1414x over previous
import functools

import jax
import jax.numpy as jnp
from jax import lax
from jax.experimental import pallas as pl
from jax.experimental.pallas import tpu as pltpu


def _se_body(x_ref, w1_ref, w2t_ref, o_ref, *, inv_hw):
    xf = x_ref[...].astype(jnp.float32)
    pooled = jnp.sum(xf, axis=1) * inv_hw
    h = lax.dot_general(
        pooled, w1_ref[...], (((1,), (1,)), ((), ())),
        preferred_element_type=jnp.float32)
    h = jnp.maximum(h, 0.0)
    s = jax.nn.sigmoid(
        jnp.dot(h, w2t_ref[...], preferred_element_type=jnp.float32))
    o_ref[...] = (xf * s[:, None, :]).astype(o_ref.dtype)


@jax.jit
def kernel(x_nchw, w1, w2):
    B, C, H, W = x_nchw.shape
    HW = H * W
    x = x_nchw.transpose(0, 2, 3, 1).reshape(B, HW, C)
    w2t = w2.T

    TB = 8
    while B % TB:
        TB -= 1

    out = pl.pallas_call(
        functools.partial(_se_body, inv_hw=1.0 / float(HW)),
        out_shape=jax.ShapeDtypeStruct((B, HW, C), x.dtype),
        grid=(B // TB,),
        in_specs=[
            pl.BlockSpec((TB, HW, C), lambda b: (b, 0, 0)),
            pl.BlockSpec(w1.shape, lambda b: (0, 0)),
            pl.BlockSpec(w2t.shape, lambda b: (0, 0)),
        ],
        out_specs=pl.BlockSpec((TB, HW, C), lambda b: (b, 0, 0)),
        compiler_params=pltpu.CompilerParams(
            dimension_semantics=("parallel",),
            vmem_limit_bytes=48 << 20,
        ),
        cost_estimate=pl.CostEstimate(
            flops=int(2 * B * C * HW + 4 * B * C * w1.shape[0]),
            transcendentals=int(B * C),
            bytes_accessed=int(2 * B * C * HW * x.dtype.itemsize),
        ),
    )(x, w1, w2t)
    return out.reshape(B, H, W, C).transpose(0, 3, 1, 2)


# speedup vs baseline: 3.6999x; 1.0602x over previous
import functools

import jax
import jax.numpy as jnp
from jax import lax
from jax.experimental import pallas as pl
from jax.experimental.pallas import tpu as pltpu


def _se_body(x_ref, w1_ref, w2t_ref, o_ref, *, inv_hw):
    xf = x_ref[...].astype(jnp.float32)
    pooled = jnp.sum(xf, axis=1) * inv_hw
    h = lax.dot_general(
        pooled, w1_ref[...], (((1,), (1,)), ((), ())),
        preferred_element_type=jnp.float32)
    h = jnp.maximum(h, 0.0)
    s = jax.nn.sigmoid(
        jnp.dot(h, w2t_ref[...], preferred_element_type=jnp.float32))
    o_ref[...] = (xf * s[:, None, :]).astype(o_ref.dtype)


@jax.jit
def kernel(x_nchw, w1, w2):
    B, C, H, W = x_nchw.shape
    HW = H * W
    x = x_nchw.transpose(0, 2, 3, 1).reshape(B, HW, C)
    w2t = w2.T

    TB = 16
    while B % TB:
        TB -= 1

    out = pl.pallas_call(
        functools.partial(_se_body, inv_hw=1.0 / float(HW)),
        out_shape=jax.ShapeDtypeStruct((B, HW, C), x.dtype),
        grid=(B // TB,),
        in_specs=[
            pl.BlockSpec((TB, HW, C), lambda b: (b, 0, 0)),
            pl.BlockSpec(w1.shape, lambda b: (0, 0)),
            pl.BlockSpec(w2t.shape, lambda b: (0, 0)),
        ],
        out_specs=pl.BlockSpec((TB, HW, C), lambda b: (b, 0, 0)),
        compiler_params=pltpu.CompilerParams(
            dimension_semantics=("parallel",),
            vmem_limit_bytes=48 << 20,
        ),
        cost_estimate=pl.CostEstimate(
            flops=int(2 * B * C * HW + 4 * B * C * w1.shape[0]),
            transcendentals=int(B * C),
            bytes_accessed=int(2 * B * C * HW * x.dtype.itemsize),
        ),
    )(x, w1, w2t)
    return out.reshape(B, H, W, C).transpose(0, 3, 1, 2)
